# Initial kernel scaffold; baseline (speedup 1.0000x reference)
import jax
import jax.numpy as jnp
from jax.experimental import pallas as pl


def kernel(encoder_outputs, syn_embeddeds, subj, obj, edge_index, W_attn, W_gcn, b_gcn, W_out, b_out):
    raise NotImplementedError("write your pallas kernel here")


# speedup vs baseline: 26.4217x; 26.4217x over previous
import functools

import jax
import jax.numpy as jnp
from jax import lax
from jax.experimental import pallas as pl
from jax.experimental.pallas import tpu as pltpu
from jax.experimental.pallas import tpu_sc as plsc

N = 10000
E = 160000
D = 256
OUT = 128


EB = 4000
NSTEPS = E // EB


def _scores_body(x0_ref, at_ref, syn_ref, scores_ref, m_ref, q_s, m_s):
    i = pl.program_id(0)

    @pl.when(i == 0)
    def _():
        q_s[...] = jnp.dot(x0_ref[...], at_ref[...],
                           preferred_element_type=jnp.float32)

    s = jnp.sum(syn_ref[...] * q_s[...], axis=1, keepdims=True)
    scores_ref[...] = s
    bm = jnp.max(s, axis=0, keepdims=True)

    @pl.when(i == 0)
    def _():
        m_s[...] = bm

    @pl.when(i > 0)
    def _():
        m_s[...] = jnp.maximum(m_s[...], bm)

    @pl.when(i == NSTEPS - 1)
    def _():
        m_ref[...] = m_s[...]


def _scores_call(x0, at, syn):
    return pl.pallas_call(
        _scores_body,
        grid=(NSTEPS,),
        in_specs=[
            pl.BlockSpec((1, D), lambda i: (0, 0)),
            pl.BlockSpec((D, D), lambda i: (0, 0)),
            pl.BlockSpec((EB, D), lambda i: (i, 0)),
        ],
        out_specs=[
            pl.BlockSpec((EB, 1), lambda i: (i, 0)),
            pl.BlockSpec((1, 1), lambda i: (0, 0)),
        ],
        out_shape=[
            jax.ShapeDtypeStruct((E, 1), jnp.float32),
            jax.ShapeDtypeStruct((1, 1), jnp.float32),
        ],
        scratch_shapes=[
            pltpu.VMEM((1, D), jnp.float32),
            pltpu.VMEM((1, 1), jnp.float32),
        ],
    )(x0, at, syn)


NC = 2
NS = 16
NW = NC * NS
CHUNK = E // NW
NVEC = -(-CHUNK // 16)
BUF = NVEC * 16


def _scatter_body(scores_hbm, src_hbm, dst_hbm, m_hbm,
                  pdeg_hbm, pu_hbm,
                  sc_v, src_v, dst_v, m_v, acc_deg, acc_u):
    wid = lax.axis_index("s") * NC + lax.axis_index("c")
    base = wid * CHUNK

    sc_v[pl.ds(BUF - 16, 16)] = jnp.full((16,), -1e30, jnp.float32)
    src_v[pl.ds(BUF - 16, 16)] = jnp.zeros((16,), jnp.int32)
    dst_v[pl.ds(BUF - 16, 16)] = jnp.zeros((16,), jnp.int32)

    pltpu.sync_copy(scores_hbm.at[pl.ds(base, CHUNK)], sc_v.at[pl.ds(0, CHUNK)])
    pltpu.sync_copy(src_hbm.at[pl.ds(base, CHUNK)], src_v.at[pl.ds(0, CHUNK)])
    pltpu.sync_copy(dst_hbm.at[pl.ds(base, CHUNK)], dst_v.at[pl.ds(0, CHUNK)])
    pltpu.sync_copy(m_hbm, m_v)

    def zero_body(i, carry):
        z = jnp.zeros((16,), jnp.float32)
        acc_deg[pl.ds(i * 16, 16)] = z
        acc_u[pl.ds(i * 16, 16)] = z
        return carry

    lax.fori_loop(0, N // 16, zero_body, 0)

    mv = m_v[...]

    def body(j, carry):
        o = j * 16
        ewu = jnp.exp(sc_v[pl.ds(o, 16)] - mv)
        dstv = dst_v[pl.ds(o, 16)]
        srcv = src_v[pl.ds(o, 16)]
        plsc.addupdate_scatter(acc_deg, [dstv], ewu)
        v0 = jnp.where(dstv == 0, ewu, jnp.zeros((16,), jnp.float32))
        plsc.addupdate_scatter(acc_u, [srcv], v0)
        return carry

    lax.fori_loop(0, NVEC, body, 0)

    pltpu.sync_copy(acc_deg, pdeg_hbm.at[wid])
    pltpu.sync_copy(acc_u, pu_hbm.at[wid])


_scatter_call = functools.partial(
    pl.kernel,
    out_type=[
        jax.ShapeDtypeStruct((NW, N), jnp.float32),
        jax.ShapeDtypeStruct((NW, N), jnp.float32),
    ],
    mesh=plsc.VectorSubcoreMesh(core_axis_name="c", subcore_axis_name="s",
                                num_cores=NC, num_subcores=NS),
    compiler_params=pltpu.CompilerParams(needs_layout_passes=False),
    scratch_types=[
        pltpu.VMEM((BUF,), jnp.float32),
        pltpu.VMEM((BUF,), jnp.int32),
        pltpu.VMEM((BUF,), jnp.int32),
        pltpu.VMEM((16,), jnp.float32),
        pltpu.VMEM((N,), jnp.float32),
        pltpu.VMEM((N,), jnp.float32),
    ],
)(_scatter_body)


def _combine_body(pdeg_ref, pu_ref, a_ref, diz0_ref):
    deg_star = jnp.sum(pdeg_ref[...], axis=0, keepdims=True)
    u_star = jnp.sum(pu_ref[...], axis=0, keepdims=True)
    z = jnp.sum(deg_star, axis=1, keepdims=True)
    deg = deg_star / z + 1.0
    diz = lax.rsqrt(deg)
    a = diz * (u_star / z)
    iota = lax.broadcasted_iota(jnp.int32, a.shape, 1)
    a = a + jnp.where(iota == 0, diz, 0.0)
    a_ref[...] = a
    diz0_ref[...] = diz[:, :1]


def _combine_call(pdeg, pu):
    return pl.pallas_call(
        _combine_body,
        out_shape=[
            jax.ShapeDtypeStruct((1, N), jnp.float32),
            jax.ShapeDtypeStruct((1, 1), jnp.float32),
        ],
    )(pdeg, pu)


NB = 2000
NBSTEPS = N // NB


def _final_body(a_ref, x_ref, diz0_ref, wgt_ref, bg_ref, subj_ref, obj_ref,
                wot_ref, bo_ref, out_ref, acc):
    i = pl.program_id(0)
    part = jnp.sum(x_ref[...] * a_ref[...], axis=0, keepdims=True)

    @pl.when(i == 0)
    def _():
        acc[...] = part

    @pl.when(i > 0)
    def _():
        acc[...] = acc[...] + part

    @pl.when(i == NBSTEPS - 1)
    def _():
        pre = acc[...]
        o0 = diz0_ref[...] * jnp.dot(pre, wgt_ref[...],
                                     preferred_element_type=jnp.float32)
        o0 = o0 + bg_ref[...]
        cat = jnp.concatenate([o0, subj_ref[...], obj_ref[...]], axis=1)
        logits = jnp.dot(cat, wot_ref[...],
                         preferred_element_type=jnp.float32) + bo_ref[...]
        ls = logits - jnp.max(logits, axis=1, keepdims=True)
        out_ref[...] = ls - jnp.log(jnp.sum(jnp.exp(ls), axis=1,
                                            keepdims=True))


def _final_call(a_col, x, diz0, wgt, bg, subj, obj, wot, bo):
    return pl.pallas_call(
        _final_body,
        grid=(NBSTEPS,),
        in_specs=[
            pl.BlockSpec((NB, 1), lambda i: (i, 0)),
            pl.BlockSpec((NB, D), lambda i: (i, 0)),
            pl.BlockSpec((1, 1), lambda i: (0, 0)),
            pl.BlockSpec((D, D), lambda i: (0, 0)),
            pl.BlockSpec((1, D), lambda i: (0, 0)),
            pl.BlockSpec((1, D), lambda i: (0, 0)),
            pl.BlockSpec((1, D), lambda i: (0, 0)),
            pl.BlockSpec((3 * D, OUT), lambda i: (0, 0)),
            pl.BlockSpec((1, OUT), lambda i: (0, 0)),
        ],
        out_specs=pl.BlockSpec((1, OUT), lambda i: (0, 0)),
        out_shape=jax.ShapeDtypeStruct((1, OUT), jnp.float32),
        scratch_shapes=[pltpu.VMEM((1, D), jnp.float32)],
    )(a_col, x, diz0, wgt, bg, subj, obj, wot, bo)


def kernel(encoder_outputs, syn_embeddeds, subj, obj, edge_index,
           W_attn, W_gcn, b_gcn, W_out, b_out):
    x0 = encoder_outputs[0:1]
    scores2, m = _scores_call(x0, W_attn.T, syn_embeddeds)
    scores = scores2.reshape(E)
    m16 = jnp.broadcast_to(m.reshape(1), (16,))
    src = edge_index[0]
    dst = edge_index[1]
    pdeg, pu = _scatter_call(scores, src, dst, m16)
    a, diz0 = _combine_call(pdeg, pu)
    out = _final_call(a.reshape(N, 1), encoder_outputs, diz0, W_gcn.T,
                      b_gcn.reshape(1, D), subj.reshape(1, D),
                      obj.reshape(1, D), W_out.T, b_out.reshape(1, OUT))
    return out


# speedup vs baseline: 30.1870x; 1.1425x over previous
import functools

import jax
import jax.numpy as jnp
from jax import lax
from jax.experimental import pallas as pl
from jax.experimental.pallas import tpu as pltpu
from jax.experimental.pallas import tpu_sc as plsc

N = 10000
E = 160000
D = 256
OUT = 128

_DOT_T = (((1,), (1,)), ((), ()))


EB = 16000
NSTEPS = E // EB


def _scores_body(x0_ref, wa_ref, syn_ref, scores_ref, m_ref, q_s, m_s):
    i = pl.program_id(0)

    @pl.when(i == 0)
    def _():
        q_s[...] = lax.dot_general(x0_ref[...], wa_ref[...], _DOT_T,
                                   preferred_element_type=jnp.float32)

    s = jnp.sum(syn_ref[...] * q_s[...], axis=1, keepdims=True)
    scores_ref[...] = s
    bm = jnp.max(s, axis=0, keepdims=True)

    @pl.when(i == 0)
    def _():
        m_s[...] = bm

    @pl.when(i > 0)
    def _():
        m_s[...] = jnp.maximum(m_s[...], bm)

    @pl.when(i == NSTEPS - 1)
    def _():
        m_ref[...] = jnp.broadcast_to(m_s[...], (1, 16))


def _scores_call(x0, wa, syn):
    return pl.pallas_call(
        _scores_body,
        grid=(NSTEPS,),
        in_specs=[
            pl.BlockSpec((1, D), lambda i: (0, 0)),
            pl.BlockSpec((D, D), lambda i: (0, 0)),
            pl.BlockSpec((EB, D), lambda i: (i, 0)),
        ],
        out_specs=[
            pl.BlockSpec((EB, 1), lambda i: (i, 0)),
            pl.BlockSpec((1, 16), lambda i: (0, 0)),
        ],
        out_shape=[
            jax.ShapeDtypeStruct((E, 1), jnp.float32),
            jax.ShapeDtypeStruct((1, 16), jnp.float32),
        ],
        scratch_shapes=[
            pltpu.VMEM((1, D), jnp.float32),
            pltpu.VMEM((1, 1), jnp.float32),
        ],
    )(x0, wa, syn)


NC = 2
NS = 16
NW = NC * NS
CHUNK = E // NW
NVEC = -(-CHUNK // 16)
BUF = NVEC * 16


def _scatter_body(scores_hbm, ei_hbm, m_hbm,
                  pdeg_hbm, pu_hbm,
                  sc_v, src_v, dst_v, m_v, acc_deg, acc_u):
    wid = lax.axis_index("s") * NC + lax.axis_index("c")
    base = wid * CHUNK

    sc_v[pl.ds(BUF - 16, 16)] = jnp.full((16,), -1e30, jnp.float32)
    src_v[pl.ds(BUF - 16, 16)] = jnp.zeros((16,), jnp.int32)
    dst_v[pl.ds(BUF - 16, 16)] = jnp.zeros((16,), jnp.int32)

    pltpu.sync_copy(scores_hbm.at[pl.ds(base, CHUNK)], sc_v.at[pl.ds(0, CHUNK)])
    pltpu.sync_copy(ei_hbm.at[pl.ds(base, CHUNK)], src_v.at[pl.ds(0, CHUNK)])
    pltpu.sync_copy(ei_hbm.at[pl.ds(E + base, CHUNK)], dst_v.at[pl.ds(0, CHUNK)])
    pltpu.sync_copy(m_hbm.at[0], m_v)

    def zero_body(i, carry):
        z = jnp.zeros((16,), jnp.float32)
        acc_deg[pl.ds(i * 16, 16)] = z
        acc_u[pl.ds(i * 16, 16)] = z
        return carry

    lax.fori_loop(0, N // 16, zero_body, 0)

    mv = m_v[...]

    def body(j, carry):
        o = j * 16
        ewu = jnp.exp(sc_v[pl.ds(o, 16)] - mv)
        dstv = dst_v[pl.ds(o, 16)]
        srcv = src_v[pl.ds(o, 16)]
        plsc.addupdate_scatter(acc_deg, [dstv], ewu)
        v0 = jnp.where(dstv == 0, ewu, jnp.zeros((16,), jnp.float32))
        plsc.addupdate_scatter(acc_u, [srcv], v0)
        return carry

    lax.fori_loop(0, NVEC, body, 0)

    pltpu.sync_copy(acc_deg, pdeg_hbm.at[wid])
    pltpu.sync_copy(acc_u, pu_hbm.at[wid])


_scatter_call = functools.partial(
    pl.kernel,
    out_type=[
        jax.ShapeDtypeStruct((NW, N), jnp.float32),
        jax.ShapeDtypeStruct((NW, N), jnp.float32),
    ],
    mesh=plsc.VectorSubcoreMesh(core_axis_name="c", subcore_axis_name="s",
                                num_cores=NC, num_subcores=NS),
    compiler_params=pltpu.CompilerParams(needs_layout_passes=False),
    scratch_types=[
        pltpu.VMEM((BUF,), jnp.float32),
        pltpu.VMEM((BUF,), jnp.int32),
        pltpu.VMEM((BUF,), jnp.int32),
        pltpu.VMEM((16,), jnp.float32),
        pltpu.VMEM((N,), jnp.float32),
        pltpu.VMEM((N,), jnp.float32),
    ],
)(_scatter_body)


def _final_body(pdeg_ref, pu_ref, x_ref, wg_ref, bg_ref, subj_ref, obj_ref,
                wo_ref, bo_ref, out_ref):
    deg_star = jnp.sum(pdeg_ref[...], axis=0, keepdims=True)
    u_star = jnp.sum(pu_ref[...], axis=0, keepdims=True)
    z = jnp.sum(deg_star, axis=1, keepdims=True)
    deg = deg_star / z + 1.0
    diz = lax.rsqrt(deg)
    a = diz * (u_star / z)
    iota = lax.broadcasted_iota(jnp.int32, a.shape, 1)
    a = a + jnp.where(iota == 0, diz, 0.0)
    pre = jnp.dot(a, x_ref[...], preferred_element_type=jnp.float32)
    o0 = diz[:, :1] * lax.dot_general(pre, wg_ref[...], _DOT_T,
                                      preferred_element_type=jnp.float32)
    o0 = o0 + bg_ref[...]
    cat = jnp.concatenate([o0, subj_ref[...], obj_ref[...]], axis=1)
    logits = lax.dot_general(cat, wo_ref[...], _DOT_T,
                             preferred_element_type=jnp.float32) + bo_ref[...]
    ls = logits - jnp.max(logits, axis=1, keepdims=True)
    out_ref[...] = ls - jnp.log(jnp.sum(jnp.exp(ls), axis=1, keepdims=True))


def _final_call(pdeg, pu, x, wg, bg, subj, obj, wo, bo):
    return pl.pallas_call(
        _final_body,
        out_shape=jax.ShapeDtypeStruct((1, OUT), jnp.float32),
    )(pdeg, pu, x, wg, bg, subj, obj, wo, bo)


def kernel(encoder_outputs, syn_embeddeds, subj, obj, edge_index,
           W_attn, W_gcn, b_gcn, W_out, b_out):
    x0 = encoder_outputs[0:1]
    scores2, m = _scores_call(x0, W_attn, syn_embeddeds)
    scores = scores2.reshape(E)
    pdeg, pu = _scatter_call(scores, edge_index.reshape(2 * E), m)
    out = _final_call(pdeg, pu, encoder_outputs, W_gcn,
                      b_gcn.reshape(1, D), subj.reshape(1, D),
                      obj.reshape(1, D), W_out, b_out.reshape(1, OUT))
    return out
